# baseline (speedup 1.0000x reference)
import functools

import jax
import jax.numpy as jnp
from jax import lax
from jax.experimental import pallas as pl
from jax.experimental.pallas import tpu as pltpu
from jax.experimental.pallas import tpu_sc as plsc

NC = 2
NS = 16
L = 16
NW = NC * NS

BATCH = 4096
SEQ = 50
N_TOKENS = BATCH * SEQ
N_SLOTS = 128
D = 64

B_PER_W = N_TOKENS // NW
ROWS_PER_W = BATCH // NW
CHUNK = 128
N_CHUNKS = B_PER_W // CHUNK


G_PER_BLK = 5
BLK = CHUNK * G_PER_BLK
N_BLK = B_PER_W // BLK
NBUF = 2


N_EXT = 2 * N_SLOTS


def _sc_body(ids_hbm, first_hbm, emb_hbm, out_hbm,
             ids_v, idx_v, first_v, bld_v, rows_v, shared, gsem, ssem):
    sid = lax.axis_index("s")
    wid = sid * NC + lax.axis_index("c")
    base = wid * B_PER_W

    pltpu.sync_copy(first_hbm.at[pl.ds(0, L)], first_v)
    lo = first_v[...] - lax.iota(jnp.int32, L)
    hi = lo + N_SLOTS

    @pl.when(sid == 0)
    def _build():
        pltpu.sync_copy(emb_hbm, bld_v.at[pl.ds(0, N_SLOTS)])
        r0 = [bld_v[0, pl.ds(k * L, L)] for k in range(D // L)]

        def rep(r, carry):
            for k in range(D // L):
                bld_v[N_SLOTS + r, pl.ds(k * L, L)] = r0[k]
            return carry

        lax.fori_loop(0, N_SLOTS, rep, 0)
        pltpu.sync_copy(bld_v, shared)

    pltpu.sync_copy(ids_hbm.at[pl.ds(wid * ROWS_PER_W, ROWS_PER_W)], ids_v)
    lanes = lax.iota(jnp.int32, L)

    def remap(g, carry):
        s = g * L + lanes
        r = lax.div(s, jnp.int32(SEQ))
        t = plsc.load_gather(ids_v, [r, s - r * SEQ])
        ok = (t >= lo) & (t < hi)
        idx_v[pl.ds(g * L, L)] = jnp.where(
            ok, t - lo, N_SLOTS + (t & (N_SLOTS - 1)))
        return carry

    lax.fori_loop(0, B_PER_W // L, remap, 0)
    plsc.subcore_barrier()

    def fire_block(blk):
        b = blk % NBUF
        for s in range(G_PER_BLK):
            g = blk * G_PER_BLK + s
            pltpu.async_copy(shared.at[idx_v.at[pl.ds(g * CHUNK, CHUNK)]],
                             rows_v.at[b, pl.ds(s * CHUNK, CHUNK)], gsem)

    def store_handle(blk):
        b = blk % NBUF
        return pltpu.make_async_copy(
            rows_v.at[b], out_hbm.at[pl.ds(base + blk * BLK, BLK)], ssem)

    fire_block(0)
    for blk in range(N_BLK):
        if blk + 1 < N_BLK:
            fire_block(blk + 1)
        for s in range(G_PER_BLK):
            g = blk * G_PER_BLK + s
            b = blk % NBUF
            pltpu.make_async_copy(
                shared.at[idx_v.at[pl.ds(g * CHUNK, CHUNK)]],
                rows_v.at[b, pl.ds(s * CHUNK, CHUNK)], gsem).wait()
        if blk >= NBUF:
            store_handle(blk - NBUF).wait()
        store_handle(blk).start()
    for blk in range(N_BLK - NBUF, N_BLK):
        store_handle(blk).wait()


@jax.jit
def _lookup(ids, input_ids, emb_weight):
    mesh = plsc.VectorSubcoreMesh(core_axis_name="c", subcore_axis_name="s",
                                  num_cores=NC, num_subcores=NS)
    f = pl.kernel(
        _sc_body,
        out_type=jax.ShapeDtypeStruct((N_TOKENS, D), jnp.float32),
        mesh=mesh,
        scratch_types=[
            pltpu.VMEM((ROWS_PER_W, SEQ), jnp.int32),
            pltpu.VMEM((B_PER_W,), jnp.int32),
            pltpu.VMEM((L,), jnp.int32),
            pltpu.VMEM((N_EXT, D), jnp.float32),
            pltpu.VMEM((NBUF, BLK, D), jnp.float32),
            pltpu.VMEM_SHARED((N_EXT, D), jnp.float32),
            pltpu.SemaphoreType.DMA,
            pltpu.SemaphoreType.DMA,
        ],
        compiler_params=pltpu.CompilerParams(use_tc_tiling_on_sc=False,
                                             needs_layout_passes=False),
    )
    return f(ids, input_ids, emb_weight)


def kernel(prompt_token_ids, input_ids, emb_weight):
    return _lookup(prompt_token_ids, input_ids, emb_weight)
